# baseline (speedup 1.0000x reference)
import jax
import jax.numpy as jnp
from jax import lax
from jax.experimental import pallas as pl
from jax.experimental.pallas import tpu as pltpu
from jax.experimental.pallas import tpu_sc as plsc

_ATOM_WEIGHTS = [1.008, 12.011, 14.007, 15.999, 18.998, 30.974, 32.06,
                 35.453, 79.904, 126.904, 10.811, 28.086, 78.971, 22.99,
                 39.098, 6.941]
_MAX_WEIGHT = 126.904
_WNORM = [w / _MAX_WEIGHT for w in _ATOM_WEIGHTS]

_B = 16384
_A = 200
_C = 16
_NW = 32
_MOLS_PER_W = _B // _NW
_MOL_WORDS = _A * _C
_P = 8
_CHUNKS = _MOLS_PER_W // _P
_PAIRS = _P // 2
_GROUPS = 2 * _A // _C
_GHALF = _GROUPS // 2


def _argmax_weight(vals, weights):
  items = list(zip(vals, weights))
  while len(items) > 1:
    nxt = []
    for i in range(0, len(items), 2):
      v1, w1 = items[i]
      v2, w2 = items[i + 1]
      gt = v2 > v1
      nxt.append((jnp.where(gt, v2, v1), jnp.where(gt, w2, w1)))
    items = nxt
  return items[0][1]


def _tec_body(x_hbm, out_hbm, buf, sums, outv):
  wid = lax.axis_index("s") * 2 + lax.axis_index("c")
  mol0 = wid * _MOLS_PER_W

  lane = lax.iota(jnp.int32, 16)
  lane16 = lane * _C
  lo_half = lane < 8
  hi8 = jnp.where(lo_half, 0, 1)
  a_bound = jnp.where(lo_half, 192 + lane, lane - 8)
  zero = jnp.zeros((16,), jnp.float32)
  wsplats = [jnp.full((16,), w, jnp.float32) for w in _WNORM]

  def pair_body(m_a, acc_a, acc_b):
    for g in range(_GROUPS):
      if g < _GHALF:
        mvec = jnp.full((16,), 0, jnp.int32) + m_a
        avec = g * _C + lane
      elif g == _GHALF:
        mvec = m_a + hi8
        avec = a_bound
      else:
        mvec = jnp.full((16,), 0, jnp.int32) + (m_a + 1)
        avec = (g * _C - _A) + lane
      vals = [plsc.load_gather(buf, [mvec, avec, jnp.full((16,), c, jnp.int32)])
              for c in range(_C)]
      w = _argmax_weight(vals, wsplats)
      if g < _GHALF:
        acc_a = acc_a + w
      elif g == _GHALF:
        acc_a = acc_a + jnp.where(lo_half, w, zero)
        acc_b = acc_b + jnp.where(lo_half, zero, w)
      else:
        acc_b = acc_b + w
    return acc_a, acc_b

  def chunk_body(ci, carry):
    pltpu.sync_copy(x_hbm.at[pl.ds(mol0 + ci * _P, _P)], buf)

    def pair_loop(p, c2):
      acc_a, acc_b = pair_body(2 * p, zero, zero)
      row = (ci * _P + 2 * p) * _C
      sums[pl.ds(row, _C)] = acc_a
      sums[pl.ds(row + _C, _C)] = acc_b
      return c2

    return lax.fori_loop(0, _PAIRS, pair_loop, carry)

  lax.fori_loop(0, _CHUNKS, chunk_body, 0)

  def red_body(mg, carry):
    base = mg * (_C * _C)
    tot = zero
    for j in range(_C):
      tot = tot + plsc.load_gather(sums, [base + j + lane16])
    outv[pl.ds(mg * _C, _C)] = tot
    return carry

  lax.fori_loop(0, _MOLS_PER_W // _C, red_body, 0)

  pltpu.sync_copy(outv, out_hbm.at[pl.ds(mol0, _MOLS_PER_W)])


_mesh = plsc.VectorSubcoreMesh(core_axis_name="c", subcore_axis_name="s")


@jax.jit
def _weight_feature(x):
  return pl.kernel(
      _tec_body,
      out_type=jax.ShapeDtypeStruct((_B,), jnp.float32),
      mesh=_mesh,
      scratch_types=[
          pltpu.VMEM((_P, _A, _C), jnp.float32),
          pltpu.VMEM((_MOLS_PER_W * _C,), jnp.float32),
          pltpu.VMEM((_MOLS_PER_W,), jnp.float32),
      ],
      compiler_params=pltpu.CompilerParams(
          needs_layout_passes=False, use_tc_tiling_on_sc=False),
  )(x)


def kernel(X):
  return _weight_feature(X).reshape(_B, 1)


# speedup vs baseline: 8.0518x; 8.0518x over previous
import jax
import jax.numpy as jnp
from jax import lax
from jax.experimental import pallas as pl
from jax.experimental.pallas import tpu as pltpu
from jax.experimental.pallas import tpu_sc as plsc

_ATOM_WEIGHTS = [1.008, 12.011, 14.007, 15.999, 18.998, 30.974, 32.06,
                 35.453, 79.904, 126.904, 10.811, 28.086, 78.971, 22.99,
                 39.098, 6.941]
_MAX_WEIGHT = 126.904
_WNORM = [w / _MAX_WEIGHT for w in _ATOM_WEIGHTS]

_B = 16384
_A = 200
_C = 16
_NW = 32
_MOLS_PER_W = _B // _NW
_KA = 8
_NCH = _A // _KA
_NG = _MOLS_PER_W // _C


def _argmax_weight(vals, weights):
  items = list(zip(vals, weights))
  while len(items) > 1:
    nxt = []
    for i in range(0, len(items), 2):
      v1, w1 = items[i]
      v2, w2 = items[i + 1]
      gt = v2 > v1
      nxt.append((jnp.where(gt, v2, v1), jnp.where(gt, w2, w1)))
    items = nxt
  return items[0][1]


def _tec_body(y_hbm, out_hbm, buf, acc_v):
  wid = lax.axis_index("s") * 2 + lax.axis_index("c")
  mol0 = wid * _MOLS_PER_W

  zero = jnp.zeros((16,), jnp.float32)
  wsplats = [jnp.full((16,), w, jnp.float32) for w in _WNORM]

  def init_body(g, carry):
    acc_v[pl.ds(g * _C, _C)] = zero
    return carry

  lax.fori_loop(0, _NG, init_body, 0)

  def chunk_body(ci, carry):
    pltpu.sync_copy(
        y_hbm.at[pl.ds(ci * _KA, _KA), :, pl.ds(mol0, _MOLS_PER_W)], buf)

    def group_body(g, c2):
      m0 = g * _C
      acc = acc_v[pl.ds(m0, _C)]
      for ai in range(_KA):
        vals = [buf[ai, c, pl.ds(m0, _C)] for c in range(_C)]
        acc = acc + _argmax_weight(vals, wsplats)
      acc_v[pl.ds(m0, _C)] = acc
      return c2

    return lax.fori_loop(0, _NG, group_body, carry)

  lax.fori_loop(0, _NCH, chunk_body, 0)

  pltpu.sync_copy(acc_v, out_hbm.at[pl.ds(mol0, _MOLS_PER_W)])


_mesh = plsc.VectorSubcoreMesh(core_axis_name="c", subcore_axis_name="s")


@jax.jit
def _weight_feature(x):
  y = jnp.transpose(x, (1, 2, 0))
  return pl.kernel(
      _tec_body,
      out_type=jax.ShapeDtypeStruct((_B,), jnp.float32),
      mesh=_mesh,
      scratch_types=[
          pltpu.VMEM((_KA, _C, _MOLS_PER_W), jnp.float32),
          pltpu.VMEM((_MOLS_PER_W,), jnp.float32),
      ],
      compiler_params=pltpu.CompilerParams(
          needs_layout_passes=False, use_tc_tiling_on_sc=True),
  )(y)


def kernel(X):
  return _weight_feature(X).reshape(_B, 1)


# speedup vs baseline: 12.7960x; 1.5892x over previous
import jax
import jax.numpy as jnp
from jax import lax
from jax.experimental import pallas as pl
from jax.experimental.pallas import tpu as pltpu
from jax.experimental.pallas import tpu_sc as plsc

_ATOM_WEIGHTS = [1.008, 12.011, 14.007, 15.999, 18.998, 30.974, 32.06,
                 35.453, 79.904, 126.904, 10.811, 28.086, 78.971, 22.99,
                 39.098, 6.941]
_MAX_WEIGHT = 126.904
_WNORM = [w / _MAX_WEIGHT for w in _ATOM_WEIGHTS]

_B = 16384
_A = 200
_C = 16
_NW = 32
_MOLS_PER_W = _B // _NW
_KA = 4
_NCH = _A // _KA
_NG = _MOLS_PER_W // _C


def _argmax_weight(vals, weights):
  items = list(zip(vals, weights))
  while len(items) > 1:
    nxt = []
    for i in range(0, len(items), 2):
      v1, w1 = items[i]
      v2, w2 = items[i + 1]
      gt = v2 > v1
      nxt.append((jnp.where(gt, v2, v1), jnp.where(gt, w2, w1)))
    items = nxt
  return items[0][1]


def _tec_body(y_hbm, out_hbm, buf0, buf1, acc_v, sem0, sem1):
  wid = lax.axis_index("s") * 2 + lax.axis_index("c")
  mol0 = wid * _MOLS_PER_W

  zero = jnp.zeros((16,), jnp.float32)
  wsplats = [jnp.full((16,), w, jnp.float32) for w in _WNORM]
  bufs = (buf0, buf1)
  sems = (sem0, sem1)

  def src(ci):
    return y_hbm.at[pl.ds(ci * _KA, _KA), :, pl.ds(mol0, _MOLS_PER_W)]

  def init_body(g, carry):
    acc_v[pl.ds(g * _C, _C)] = zero
    return carry

  lax.fori_loop(0, _NG, init_body, 0)

  def compute(buf):
    def group_body(g, c2):
      m0 = g * _C
      acc = acc_v[pl.ds(m0, _C)]
      for ai in range(_KA):
        vals = [buf[ai, c, pl.ds(m0, _C)] for c in range(_C)]
        acc = acc + _argmax_weight(vals, wsplats)
      acc_v[pl.ds(m0, _C)] = acc
      return c2

    lax.fori_loop(0, _NG, group_body, 0)

  pltpu.async_copy(src(0), buf0, sem0)

  def pair_body(c2, carry):
    ci = 2 * c2
    for b in range(2):
      pltpu.make_async_copy(src(ci + b), bufs[b], sems[b]).wait()
      nxt = ci + b + 1

      @pl.when(nxt < _NCH)
      def _():
        pltpu.async_copy(src(nxt), bufs[1 - b], sems[1 - b])

      compute(bufs[b])
    return carry

  lax.fori_loop(0, _NCH // 2, pair_body, 0)

  pltpu.sync_copy(acc_v, out_hbm.at[pl.ds(mol0, _MOLS_PER_W)])


_mesh = plsc.VectorSubcoreMesh(core_axis_name="c", subcore_axis_name="s")


@jax.jit
def _weight_feature(x):
  y = jnp.transpose(x, (1, 2, 0))
  return pl.kernel(
      _tec_body,
      out_type=jax.ShapeDtypeStruct((_B,), jnp.float32),
      mesh=_mesh,
      scratch_types=[
          pltpu.VMEM((_KA, _C, _MOLS_PER_W), jnp.float32),
          pltpu.VMEM((_KA, _C, _MOLS_PER_W), jnp.float32),
          pltpu.VMEM((_MOLS_PER_W,), jnp.float32),
          pltpu.SemaphoreType.DMA,
          pltpu.SemaphoreType.DMA,
      ],
      compiler_params=pltpu.CompilerParams(
          needs_layout_passes=False, use_tc_tiling_on_sc=True),
  )(y)


def kernel(X):
  return _weight_feature(X).reshape(_B, 1)
